# baseline (speedup 1.0000x reference)
import jax
import jax.numpy as jnp
from jax import lax
from jax.experimental import pallas as pl
from jax.experimental.pallas import tpu as pltpu
from jax.experimental.pallas import tpu_sc as plsc

VOCAB = 8192
D = 8192
B = 8192
B_SC = 6144
B_TC = B - B_SC
NW = 32
B_PER_W = B_SC // NW
CH = 4
NCHUNK = B_PER_W // CH
NPAIR = NCHUNK // 2
RPB = 16


def _sc_body(idx_hbm, table_hbm, out_hbm, idx_v, rows_v, g0, g1, s0, s1):
    cid = lax.axis_index("c")
    sid = lax.axis_index("s")
    wid = sid * 2 + cid
    base = wid * B_PER_W

    pltpu.sync_copy(idx_hbm.at[wid], idx_v)

    def gather(c, buf, sem):
        return pltpu.make_async_copy(
            table_hbm.at[idx_v.at[c]], rows_v.at[buf], sem)

    def scatter(c, buf, sem):
        return pltpu.make_async_copy(
            rows_v.at[buf], out_hbm.at[pl.ds(base + c * CH, CH)], sem)

    gather(0, 0, g0).start()

    def pair(i, carry):
        c0 = 2 * i
        c1 = c0 + 1

        @pl.when(i > 0)
        def _():
            scatter(c0 - 1, 1, s1).wait()

        gather(c1, 1, g1).start()
        gather(c0, 0, g0).wait()
        scatter(c0, 0, s0).start()

        @pl.when(i < NPAIR - 1)
        def _():
            scatter(c0, 0, s0).wait()
            gather(c0 + 2, 0, g0).start()

        gather(c1, 1, g1).wait()
        scatter(c1, 1, s1).start()
        return carry

    lax.fori_loop(0, NPAIR, pair, 0)

    scatter(NCHUNK - 2, 0, s0).wait()
    scatter(NCHUNK - 1, 1, s1).wait()


def _sc_gather(idx, table):
    mesh = plsc.VectorSubcoreMesh(core_axis_name="c", subcore_axis_name="s")
    return pl.kernel(
        _sc_body,
        mesh=mesh,
        out_type=jax.ShapeDtypeStruct((B_SC, D), jnp.float32),
        scratch_types=[
            pltpu.VMEM((NCHUNK, CH), jnp.int32),
            pltpu.VMEM((2, CH, D), jnp.float32),
            pltpu.SemaphoreType.DMA,
            pltpu.SemaphoreType.DMA,
            pltpu.SemaphoreType.DMA,
            pltpu.SemaphoreType.DMA,
        ],
    )(idx.reshape(NW, NCHUNK, CH), table)


def _tc_body(idx_ref, table_hbm, out_ref, sem):
    i = pl.program_id(0)
    for j in range(RPB):
        pltpu.make_async_copy(
            table_hbm.at[pl.ds(idx_ref[i * RPB + j], 1)],
            out_ref.at[pl.ds(j, 1)],
            sem,
        ).start()
    for j in range(RPB):
        pltpu.make_async_copy(
            table_hbm.at[pl.ds(idx_ref[i * RPB + j], 1)],
            out_ref.at[pl.ds(j, 1)],
            sem,
        ).wait()


def _tc_gather(idx, table):
    return pl.pallas_call(
        _tc_body,
        grid_spec=pltpu.PrefetchScalarGridSpec(
            num_scalar_prefetch=1,
            grid=(B_TC // RPB,),
            in_specs=[pl.BlockSpec(memory_space=pl.ANY)],
            out_specs=pl.BlockSpec((RPB, D), lambda i, idx_ref: (i, 0)),
            scratch_shapes=[pltpu.SemaphoreType.DMA],
        ),
        out_shape=jax.ShapeDtypeStruct((B_TC, D), jnp.float32),
    )(idx, table)


@jax.jit
def kernel(x, table):
    flat = x.reshape(B).astype(jnp.int32)
    out_sc = _sc_gather(flat[:B_SC], table)
    out_tc = _tc_gather(flat[B_SC:], table)
    out = jnp.concatenate([out_sc, out_tc], axis=0)
    return out.reshape(x.shape[0], x.shape[1], D)


# speedup vs baseline: 2.0830x; 2.0830x over previous
import jax
import jax.numpy as jnp
from jax import lax
from jax.experimental import pallas as pl
from jax.experimental.pallas import tpu as pltpu
from jax.experimental.pallas import tpu_sc as plsc

VOCAB = 8192
D = 8192
B = 8192
NW = 32
B_PER_W = B // NW
CH = 2
NCHUNK = B_PER_W // CH
NBUF = 4
NQUAD = NCHUNK // NBUF


def _sc_body(idx_hbm, table_hbm, out_hbm, idx_v, rows_v,
             g0, g1, g2, g3, s0, s1, s2, s3):
    cid = lax.axis_index("c")
    sid = lax.axis_index("s")
    wid = sid * 2 + cid
    base = wid * B_PER_W
    gsem = (g0, g1, g2, g3)
    ssem = (s0, s1, s2, s3)

    pltpu.sync_copy(idx_hbm.at[wid], idx_v)

    def gather(c, buf, sem):
        return pltpu.make_async_copy(
            table_hbm.at[idx_v.at[c]], rows_v.at[buf], sem)

    def scatter(c, buf, sem):
        return pltpu.make_async_copy(
            rows_v.at[buf], out_hbm.at[pl.ds(base + c * CH, CH)], sem)

    for b in range(NBUF):
        gather(b, b, gsem[b]).start()

    def quad(i, carry):
        for b in range(NBUF):
            c = NBUF * i + b
            gather(c, b, gsem[b]).wait()
            scatter(c, b, ssem[b]).start()
            nb = (b + 3) % NBUF

            @pl.when((c >= 1) & (c <= NCHUNK - 4))
            def _():
                scatter(c - 1, nb, ssem[nb]).wait()
                gather(c + 3, nb, gsem[nb]).start()
        return carry

    lax.fori_loop(0, NQUAD, quad, 0)

    for k in range(NBUF):
        c = NCHUNK - NBUF + k
        scatter(c, c % NBUF, ssem[c % NBUF]).wait()


@jax.jit
def kernel(x, table):
    idx = x.reshape(NW, NCHUNK, CH).astype(jnp.int32)
    mesh = plsc.VectorSubcoreMesh(core_axis_name="c", subcore_axis_name="s")
    out = pl.kernel(
        _sc_body,
        mesh=mesh,
        out_type=jax.ShapeDtypeStruct((B, D), jnp.float32),
        scratch_types=[
            pltpu.VMEM((NCHUNK, CH), jnp.int32),
            pltpu.VMEM((NBUF, CH, D), jnp.float32),
            pltpu.SemaphoreType.DMA,
            pltpu.SemaphoreType.DMA,
            pltpu.SemaphoreType.DMA,
            pltpu.SemaphoreType.DMA,
            pltpu.SemaphoreType.DMA,
            pltpu.SemaphoreType.DMA,
            pltpu.SemaphoreType.DMA,
            pltpu.SemaphoreType.DMA,
        ],
    )(idx, table)
    return out.reshape(x.shape[0], x.shape[1], D)


# speedup vs baseline: 2.1022x; 1.0092x over previous
import jax
import jax.numpy as jnp
from jax import lax
from jax.experimental import pallas as pl
from jax.experimental.pallas import tpu as pltpu
from jax.experimental.pallas import tpu_sc as plsc

VOCAB = 8192
D = 8192
B = 8192
NW = 32
B_PER_W = B // NW
CH = 4
NCHUNK = B_PER_W // CH
NPAIR = NCHUNK // 2


def _sc_body(idx_hbm, table_hbm, out_hbm, idx_v, rows_v, g0, g1, s0, s1):
    cid = lax.axis_index("c")
    sid = lax.axis_index("s")
    wid = sid * 2 + cid
    base = wid * B_PER_W

    pltpu.sync_copy(idx_hbm.at[wid], idx_v)

    def gather(c, buf, sem):
        return pltpu.make_async_copy(
            table_hbm.at[idx_v.at[c]], rows_v.at[buf], sem)

    def scatter(c, buf, sem):
        return pltpu.make_async_copy(
            rows_v.at[buf], out_hbm.at[pl.ds(base + c * CH, CH)], sem)

    gather(0, 0, g0).start()

    def pair(i, carry):
        c0 = 2 * i
        c1 = c0 + 1

        @pl.when(i > 0)
        def _():
            scatter(c0 - 1, 1, s1).wait()

        gather(c1, 1, g1).start()

        gather(c0, 0, g0).wait()
        scatter(c0, 0, s0).start()

        @pl.when(i < NPAIR - 1)
        def _():
            scatter(c0, 0, s0).wait()
            gather(c0 + 2, 0, g0).start()

        gather(c1, 1, g1).wait()
        scatter(c1, 1, s1).start()
        return carry

    lax.fori_loop(0, NPAIR, pair, 0)

    scatter(NCHUNK - 2, 0, s0).wait()
    scatter(NCHUNK - 1, 1, s1).wait()


@jax.jit
def kernel(x, table):
    idx = x.reshape(NW, NCHUNK, CH).astype(jnp.int32)
    mesh = plsc.VectorSubcoreMesh(core_axis_name="c", subcore_axis_name="s")
    out = pl.kernel(
        _sc_body,
        mesh=mesh,
        out_type=jax.ShapeDtypeStruct((B, D), jnp.float32),
        scratch_types=[
            pltpu.VMEM((NCHUNK, CH), jnp.int32),
            pltpu.VMEM((2, CH, D), jnp.float32),
            pltpu.SemaphoreType.DMA,
            pltpu.SemaphoreType.DMA,
            pltpu.SemaphoreType.DMA,
            pltpu.SemaphoreType.DMA,
        ],
    )(idx, table)
    return out.reshape(x.shape[0], x.shape[1], D)
